# baseline (speedup 1.0000x reference)
import functools

import jax
import jax.numpy as jnp
from jax import lax
from jax.experimental import pallas as pl
from jax.experimental.pallas import tpu as pltpu
from jax.experimental.pallas import tpu_sc as plsc

_NC, _NS, _L = 2, 16, 16
_NW = _NC * _NS

_VOCAB = 1000000
_D = 64
_B = 16384
_BPW = _B // _NW
_IC = 128

_BLK = 2048
_NB = -(-_VOCAB // _BLK)
_VPAD = _NB * _BLK
_SPT = _VPAD // _NS


def _rs_body(a_ref, rs_ref):
    x = a_ref[...]
    ss = jnp.sum(x * x, axis=0, keepdims=True)
    rs_ref[...] = 1.0 / jnp.sqrt(jnp.maximum(ss, 1e-30))


_rs_call = pl.pallas_call(
    _rs_body,
    grid=(_NB,),
    in_specs=[pl.BlockSpec((_D, _BLK), lambda i: (0, i))],
    out_specs=pl.BlockSpec((1, _BLK), lambda i: (0, i)),
    out_shape=jax.ShapeDtypeStruct((1, _VPAD), jnp.float32),
)


_mesh = plsc.VectorSubcoreMesh(core_axis_name="c", subcore_axis_name="s")


@functools.partial(
    pl.kernel,
    out_type=jax.ShapeDtypeStruct((_NC, _VPAD), jnp.float32),
    mesh=_mesh,
    compiler_params=pltpu.CompilerParams(needs_layout_passes=False),
    scratch_types=[
        pltpu.VMEM((_BPW,), jnp.int32),
        pltpu.VMEM((_BPW,), jnp.float32),
        pltpu.VMEM((_SPT,), jnp.float32),
        pltpu.VMEM_SHARED((_VPAD,), jnp.float32),
    ],
)
def _hist(idx_hbm, out_hbm, idx_v, ones_v, stage_v, counts_sp):
    core = lax.axis_index("c")
    sid = lax.axis_index("s")
    wid = sid * _NC + core
    base = wid * _BPW
    pltpu.sync_copy(idx_hbm.at[pl.ds(base, _BPW)], idx_v)

    def fill(k, carry):
        sl = pl.ds(k * _L, _L)
        ones_v[sl] = jnp.full((_L,), 1.0, jnp.float32)
        stage_v[sl] = jnp.zeros((_L,), jnp.float32)
        return carry

    lax.fori_loop(0, _BPW // _L, fill, 0)
    lax.fori_loop(_BPW // _L, _SPT // _L,
                  lambda k, c: (stage_v.__setitem__(pl.ds(k * _L, _L),
                                                    jnp.zeros((_L,),
                                                              jnp.float32)),
                                c)[1], 0)

    pltpu.sync_copy(stage_v, counts_sp.at[pl.ds(sid * _SPT, _SPT)])
    plsc.subcore_barrier()

    for c in range(_BPW // _IC):
        pltpu.sync_copy(ones_v.at[pl.ds(c * _IC, _IC)],
                        counts_sp.at[idx_v.at[pl.ds(c * _IC, _IC)]],
                        add=True)
    plsc.subcore_barrier()

    pltpu.sync_copy(counts_sp.at[pl.ds(sid * _SPT, _SPT)], stage_v)
    pltpu.sync_copy(stage_v, out_hbm.at[core, pl.ds(sid * _SPT, _SPT)])


def _acc_body(a_ref, h_ref, rs_ref, out_ref):
    i = pl.program_id(0)

    @pl.when(i == 0)
    def _():
        out_ref[...] = jnp.zeros_like(out_ref)

    x = a_ref[...]
    v = (h_ref[0, :] + h_ref[1, :]) * rs_ref[0, :]
    col = jax.lax.broadcasted_iota(jnp.int32, (_D, _BLK), 1) + i * _BLK
    p = jnp.where(col < _VOCAB, x * v[None, :], 0.0)
    out_ref[...] += jnp.sum(p, axis=1, keepdims=True).reshape(1, _D)


_acc_call = pl.pallas_call(
    _acc_body,
    grid=(_NB,),
    in_specs=[
        pl.BlockSpec((_D, _BLK), lambda i: (0, i)),
        pl.BlockSpec((_NC, _BLK), lambda i: (0, i)),
        pl.BlockSpec((1, _BLK), lambda i: (0, i)),
    ],
    out_specs=pl.BlockSpec((1, _D), lambda i: (0, 0)),
    out_shape=jax.ShapeDtypeStruct((1, _D), jnp.float32),
)


def kernel(input_indices, weights):
    a = weights.T
    rs = _rs_call(a)
    hist = _hist(input_indices.astype(jnp.int32))
    return _acc_call(a, hist, rs)


# speedup vs baseline: 1.9276x; 1.9276x over previous
import functools

import jax
import jax.numpy as jnp
from jax import lax
from jax.experimental import pallas as pl
from jax.experimental.pallas import tpu as pltpu
from jax.experimental.pallas import tpu_sc as plsc

_NC, _NS, _L = 2, 16, 16
_NW = _NC * _NS

_VOCAB = 1000000
_D = 64
_B = 16384
_BPW = _B // _NW
_IC = 128

_BLK = 65536
_NB = -(-_VOCAB // _BLK)
_VPAD = _NB * _BLK
_SPT = _VPAD // _NS
_NJ = _D // 8


def _rs_body(m_ref, a_ref, rs_ref):
    j = pl.program_id(1)
    x = a_ref[...]
    p = jnp.sum(x * x, axis=0, keepdims=True)

    @pl.when(j == 0)
    def _():
        rs_ref[...] = p

    @pl.when(j > 0)
    def _():
        rs_ref[...] += p

    @pl.when(j == _NJ - 1)
    def _():
        rs_ref[...] = jnp.where(
            m_ref[...] > 0,
            1.0 / jnp.sqrt(jnp.maximum(rs_ref[...], 1e-30)),
            0.0)


_rs_call = pl.pallas_call(
    _rs_body,
    grid=(_NB, _NJ),
    in_specs=[
        pl.BlockSpec((1, _BLK), lambda i, j: (0, i)),
        pl.BlockSpec((8, _BLK), lambda i, j: (j, i)),
    ],
    out_specs=pl.BlockSpec((1, _BLK), lambda i, j: (0, i)),
    out_shape=jax.ShapeDtypeStruct((1, _VPAD), jnp.float32),
)


_mesh = plsc.VectorSubcoreMesh(core_axis_name="c", subcore_axis_name="s")

_BPT = _B // _NS
_H = _VPAD // _NC
_SPT = _H // _NS
_SENT = 0x7FFFFFFF


@functools.partial(
    pl.kernel,
    out_type=jax.ShapeDtypeStruct((_NC, _H), jnp.float32),
    mesh=_mesh,
    compiler_params=pltpu.CompilerParams(needs_layout_passes=False),
    scratch_types=[
        pltpu.VMEM((_BPT,), jnp.int32),
        pltpu.VMEM((_BPT,), jnp.int32),
        pltpu.VMEM((_BPT,), jnp.float32),
        pltpu.VMEM((_SPT,), jnp.float32),
        pltpu.VMEM_SHARED((_H,), jnp.float32),
    ],
)
def _hist(idx_hbm, out_hbm, idx_v, idx2_v, ones_v, stage_v, counts_sp):
    core = lax.axis_index("c")
    sid = lax.axis_index("s")
    base = sid * _BPT
    pltpu.sync_copy(idx_hbm.at[pl.ds(base, _BPT)], idx_v)
    lo = core * _H

    def fill(k, carry):
        sl = pl.ds(k * _L, _L)
        iv = idx_v[sl] - lo
        inhalf = jnp.logical_and(iv >= 0, iv < _H)
        idx2_v[sl] = jnp.where(inhalf, iv, _SENT)
        ones_v[sl] = jnp.full((_L,), 1.0, jnp.float32)
        return carry

    lax.fori_loop(0, _BPT // _L, fill, 0)
    lax.fori_loop(0, _SPT // _L,
                  lambda k, c: (stage_v.__setitem__(pl.ds(k * _L, _L),
                                                    jnp.zeros((_L,),
                                                              jnp.float32)),
                                c)[1], 0)

    pltpu.sync_copy(stage_v, counts_sp.at[pl.ds(sid * _SPT, _SPT)])
    plsc.subcore_barrier()

    for c in range(_BPT // _IC):
        pltpu.sync_copy(
            ones_v.at[pl.ds(c * _IC, _IC)],
            counts_sp.at[plsc.Indices(idx2_v.at[pl.ds(c * _IC, _IC)],
                                      ignored_value=_SENT)],
            add=True)
    plsc.subcore_barrier()

    pltpu.sync_copy(counts_sp.at[pl.ds(sid * _SPT, _SPT)], stage_v)
    pltpu.sync_copy(stage_v, out_hbm.at[core, pl.ds(sid * _SPT, _SPT)])


def _acc_body(m_ref, a_ref, h_ref, rs_ref, out_ref):
    i = pl.program_id(0)
    j = pl.program_id(1)

    @pl.when(jnp.logical_and(i == 0, j == 0))
    def _():
        out_ref[...] = jnp.zeros_like(out_ref)

    x = a_ref[...]
    v = h_ref[...] * rs_ref[...]
    p = jnp.where(m_ref[...] > 0, x * v, 0.0)
    s8 = jnp.sum(p, axis=1)
    out_ref[pl.ds(j, 1), :] += s8[None, :]


_acc_call = pl.pallas_call(
    _acc_body,
    grid=(_NB, _NJ),
    in_specs=[
        pl.BlockSpec((1, _BLK), lambda i, j: (0, i)),
        pl.BlockSpec((8, _BLK), lambda i, j: (j, i)),
        pl.BlockSpec((1, _BLK), lambda i, j: (0, i)),
        pl.BlockSpec((1, _BLK), lambda i, j: (0, i)),
    ],
    out_specs=pl.BlockSpec((_NJ, 8), lambda i, j: (0, 0)),
    out_shape=jax.ShapeDtypeStruct((_NJ, 8), jnp.float32),
)


def kernel(input_indices, weights):
    a = weights.T
    valid = (jnp.arange(_VPAD) < _VOCAB).astype(jnp.float32).reshape(1, -1)
    rs = _rs_call(valid, a)
    hist = _hist(input_indices.astype(jnp.int32)).reshape(1, _VPAD)
    out8 = _acc_call(valid, a, hist, rs)
    return out8.reshape(1, _D)


# speedup vs baseline: 4.8809x; 2.5321x over previous
import functools

import jax
import jax.numpy as jnp
from jax import lax
from jax.experimental import pallas as pl
from jax.experimental.pallas import tpu as pltpu
from jax.experimental.pallas import tpu_sc as plsc

_NC, _NS, _L = 2, 16, 16
_NW = _NC * _NS

_VOCAB = 1000000
_D = 64
_B = 16384
_BPW = _B // _NW
_IC = 128

_BLK = 65536
_NB = -(-_VOCAB // _BLK)
_VPAD = _NB * _BLK
_SPT = _VPAD // _NS
_NJ = _D // 8


_SUB = 8192
_NSUB = _BLK // _SUB


_mesh = plsc.VectorSubcoreMesh(core_axis_name="c", subcore_axis_name="s")

_BPT = _B // _NS
_H = _VPAD // _NC
_SPT = _H // _NS
_SENT = 0x7FFFFFFF


@functools.partial(
    pl.kernel,
    out_type=jax.ShapeDtypeStruct((_NC, _H), jnp.float32),
    mesh=_mesh,
    compiler_params=pltpu.CompilerParams(needs_layout_passes=False),
    scratch_types=[
        pltpu.VMEM((_BPT,), jnp.int32),
        pltpu.VMEM((_BPT,), jnp.int32),
        pltpu.VMEM((_BPT,), jnp.float32),
        pltpu.VMEM((_SPT,), jnp.float32),
        pltpu.VMEM_SHARED((_H,), jnp.float32),
    ],
)
def _hist(idx_hbm, out_hbm, idx_v, idx2_v, ones_v, stage_v, counts_sp):
    core = lax.axis_index("c")
    sid = lax.axis_index("s")
    base = sid * _BPT
    pltpu.sync_copy(idx_hbm.at[pl.ds(base, _BPT)], idx_v)
    lo = core * _H

    def fill(k, carry):
        sl = pl.ds(k * _L, _L)
        iv = idx_v[sl] - lo
        inhalf = jnp.logical_and(iv >= 0, iv < _H)
        idx2_v[sl] = jnp.where(inhalf, iv, _SENT)
        ones_v[sl] = jnp.full((_L,), 1.0, jnp.float32)
        return carry

    lax.fori_loop(0, _BPT // _L, fill, 0)
    lax.fori_loop(0, _SPT // _L,
                  lambda k, c: (stage_v.__setitem__(pl.ds(k * _L, _L),
                                                    jnp.zeros((_L,),
                                                              jnp.float32)),
                                c)[1], 0)

    pltpu.sync_copy(stage_v, counts_sp.at[pl.ds(sid * _SPT, _SPT)])
    plsc.subcore_barrier()

    for c in range(_BPT // _IC):
        pltpu.sync_copy(
            ones_v.at[pl.ds(c * _IC, _IC)],
            counts_sp.at[plsc.Indices(idx2_v.at[pl.ds(c * _IC, _IC)],
                                      ignored_value=_SENT)],
            add=True)
    plsc.subcore_barrier()

    pltpu.sync_copy(counts_sp.at[pl.ds(sid * _SPT, _SPT)], stage_v)
    pltpu.sync_copy(stage_v, out_hbm.at[core, pl.ds(sid * _SPT, _SPT)])


def _acc_body(m_ref, a_ref, h_ref, out_ref):
    i = pl.program_id(0)

    @pl.when(i == 0)
    def _():
        out_ref[...] = jnp.zeros_like(out_ref)

    sss = []
    for k in range(_NSUB):
        xk = a_ref[:, pl.ds(k * _SUB, _SUB)]
        sss.append(jnp.sum(xk * xk, axis=0, keepdims=True))
    part = jnp.zeros((1, _D), jnp.float32)
    for k in range(_NSUB):
        sl = pl.ds(k * _SUB, _SUB)
        rs = jnp.where(m_ref[:, sl] > 0,
                       1.0 / jnp.sqrt(jnp.maximum(sss[k], 1e-30)),
                       0.0)
        v = h_ref[:, sl] * rs
        xk = a_ref[:, sl]
        part = part + jnp.sum(xk * v, axis=1).reshape(1, _D)
    out_ref[...] += part


_acc_call = pl.pallas_call(
    _acc_body,
    grid=(_NB,),
    in_specs=[
        pl.BlockSpec((1, _BLK), lambda i: (0, i)),
        pl.BlockSpec((_D, _BLK), lambda i: (0, i)),
        pl.BlockSpec((1, _BLK), lambda i: (0, i)),
    ],
    out_specs=pl.BlockSpec((1, _D), lambda i: (0, 0)),
    out_shape=jax.ShapeDtypeStruct((1, _D), jnp.float32),
    compiler_params=pltpu.CompilerParams(vmem_limit_bytes=100 * 1024 * 1024),
)


def kernel(input_indices, weights):
    a = weights.T
    valid = (jnp.arange(_VPAD) < _VOCAB).astype(jnp.float32).reshape(1, -1)
    hist = _hist(input_indices.astype(jnp.int32)).reshape(1, _VPAD)
    return _acc_call(valid, a, hist)


# speedup vs baseline: 5.2885x; 1.0835x over previous
import functools

import jax
import jax.numpy as jnp
from jax import lax
from jax.experimental import pallas as pl
from jax.experimental.pallas import tpu as pltpu
from jax.experimental.pallas import tpu_sc as plsc

_NC, _NS, _L = 2, 16, 16
_NW = _NC * _NS

_VOCAB = 1000000
_D = 64
_B = 16384
_BPW = _B // _NW
_IC = 128

_BLK = 65536
_NB = -(-_VOCAB // _BLK)
_VPAD = _NB * _BLK
_SPT = _VPAD // _NS
_NJ = _D // 8


_SUB = 8192
_NSUB = _BLK // _SUB


_mesh = plsc.VectorSubcoreMesh(core_axis_name="c", subcore_axis_name="s")

_BPT = _B // _NS
_H = _VPAD // _NC
_SPT = _H // _NS
_SENT = 0x7FFFFFFF


@functools.partial(
    pl.kernel,
    out_type=jax.ShapeDtypeStruct((1, _VPAD), jnp.float32),
    mesh=_mesh,
    compiler_params=pltpu.CompilerParams(needs_layout_passes=False),
    scratch_types=[
        pltpu.VMEM((_BPT,), jnp.int32),
        pltpu.VMEM((_BPT,), jnp.int32),
        pltpu.VMEM((_BPT,), jnp.float32),
        pltpu.VMEM((_SPT,), jnp.float32),
        pltpu.VMEM_SHARED((_H,), jnp.float32),
    ],
)
def _hist(idx_hbm, out_hbm, idx_v, idx2_v, ones_v, stage_v, counts_sp):
    core = lax.axis_index("c")
    sid = lax.axis_index("s")
    base = sid * _BPT
    pltpu.sync_copy(idx_hbm.at[pl.ds(base, _BPT)], idx_v)
    lo = core * _H

    def fill(k, carry):
        sl = pl.ds(k * _L, _L)
        iv = idx_v[sl] - lo
        inhalf = jnp.logical_and(iv >= 0, iv < _H)
        idx2_v[sl] = jnp.where(inhalf, iv, _SENT)
        ones_v[sl] = jnp.full((_L,), 1.0, jnp.float32)
        return carry

    lax.fori_loop(0, _BPT // _L, fill, 0)
    lax.fori_loop(0, _SPT // _L,
                  lambda k, c: (stage_v.__setitem__(pl.ds(k * _L, _L),
                                                    jnp.zeros((_L,),
                                                              jnp.float32)),
                                c)[1], 0)

    pltpu.sync_copy(stage_v, counts_sp.at[pl.ds(sid * _SPT, _SPT)])
    plsc.subcore_barrier()

    for c in range(_BPT // _IC):
        pltpu.sync_copy(
            ones_v.at[pl.ds(c * _IC, _IC)],
            counts_sp.at[plsc.Indices(idx2_v.at[pl.ds(c * _IC, _IC)],
                                      ignored_value=_SENT)],
            add=True)
    plsc.subcore_barrier()

    pltpu.sync_copy(counts_sp.at[pl.ds(sid * _SPT, _SPT)], stage_v)
    pltpu.sync_copy(stage_v, out_hbm.at[0, pl.ds(lo + sid * _SPT, _SPT)])


def _acc_body(a_ref, h_ref, out_ref):
    i = pl.program_id(0)

    @pl.when(i == 0)
    def _():
        out_ref[...] = jnp.zeros_like(out_ref)

    sss = []
    for k in range(_NSUB):
        xk = a_ref[:, pl.ds(k * _SUB, _SUB)]
        sss.append(jnp.sum(xk * xk, axis=0, keepdims=True))
    part = jnp.zeros((1, _D), jnp.float32)
    for k in range(_NSUB):
        sl = pl.ds(k * _SUB, _SUB)
        h = h_ref[:, sl]
        v = jnp.where(h > 0,
                      h / jnp.sqrt(jnp.maximum(sss[k], 1e-30)),
                      0.0)
        xk = a_ref[:, sl]
        part = part + jnp.sum(xk * v, axis=1).reshape(1, _D)
    out_ref[...] += part


_acc_call = pl.pallas_call(
    _acc_body,
    grid=(_NB,),
    in_specs=[
        pl.BlockSpec((_D, _BLK), lambda i: (0, i)),
        pl.BlockSpec((1, _BLK), lambda i: (0, i)),
    ],
    out_specs=pl.BlockSpec((1, _D), lambda i: (0, 0)),
    out_shape=jax.ShapeDtypeStruct((1, _D), jnp.float32),
    compiler_params=pltpu.CompilerParams(vmem_limit_bytes=100 * 1024 * 1024),
)


def kernel(input_indices, weights):
    a = weights.T
    hist = _hist(input_indices.astype(jnp.int32))
    return _acc_call(a, hist)
